# baseline (speedup 1.0000x reference)
import jax
import jax.numpy as jnp
from jax import lax
from jax.experimental import pallas as pl
from jax.experimental.pallas import tpu as pltpu
from jax.experimental.pallas import tpu_sc as plsc

V = 10000
F = 128
FH = F // 2
E = 320000
NC = 2
NS = 16
CH = 128
NCH = 160
EPT = CH * NCH
EPAD = NS * EPT
VP = 10112
RPT = VP // NS


NB = 4


def _sc_scatter_body(x_hbm, src_hbm, dst_hbm, zero_hbm, out_hbm,
                     src_v, dst_v, rows_v,
                     g0, g1, g2, g3, s0, s1, s2, s3, acc):
    c = lax.axis_index("c")
    s = lax.axis_index("s")
    gsems = (g0, g1, g2, g3)
    ssems = (s0, s1, s2, s3)
    xc = x_hbm.at[c]

    def slot(j, b, wait_s, issue_g, jm2=None, jp2=None):
        pltpu.make_async_copy(xc.at[src_v.at[j]], rows_v.at[b],
                              gsems[b]).wait()
        pltpu.async_copy(rows_v.at[b], acc.at[dst_v.at[j]], ssems[b],
                         add=True)
        b2 = (b + 2) % NB
        if wait_s:
            pltpu.make_async_copy(rows_v.at[b2], acc.at[dst_v.at[jm2]],
                                  ssems[b2]).wait()
        if issue_g:
            pltpu.async_copy(xc.at[src_v.at[jp2]], rows_v.at[b2], gsems[b2])

    pltpu.sync_copy(zero_hbm.at[pl.ds(s * RPT, RPT)],
                    acc.at[pl.ds(s * RPT, RPT)])
    pltpu.sync_copy(src_hbm.at[s], src_v)
    pltpu.sync_copy(dst_hbm.at[s], dst_v)
    for b in range(2):
        pltpu.async_copy(xc.at[src_v.at[b]], rows_v.at[b], gsems[b])
    plsc.subcore_barrier()

    for b in range(NB):
        slot(b, b, wait_s=(b >= 2), issue_g=True, jm2=b - 2, jp2=b + 2)

    @pl.loop(1, NCH // NB - 1)
    def _(i):
        for b in range(NB):
            j = i * NB + b
            slot(j, b, wait_s=True, issue_g=True, jm2=j - 2, jp2=j + 2)

    base = NCH - NB
    for b in range(NB):
        slot(base + b, b, wait_s=True, issue_g=(b < 2),
             jm2=base + b - 2, jp2=base + b + 2)
    for b in (2, 3):
        pltpu.make_async_copy(rows_v.at[b], acc.at[dst_v.at[base + b]],
                              ssems[b]).wait()
    plsc.subcore_barrier()
    pltpu.sync_copy(acc.at[pl.ds(s * RPT, RPT)],
                    out_hbm.at[c, pl.ds(s * RPT, RPT)])


def _tc_combine_body(x_ref, p_ref, wst_ref, wnt_ref, b_ref, g_ref, be_ref,
                     o_ref):
    x = x_ref[...]
    agg = jnp.concatenate([p_ref[0], p_ref[1]], axis=-1)
    h = (jnp.dot(x, wst_ref[...], preferred_element_type=jnp.float32)
         + jnp.dot(agg, wnt_ref[...], preferred_element_type=jnp.float32)
         + b_ref[...])
    mean = jnp.mean(h, axis=0, keepdims=True)
    d = h - mean
    var = jnp.mean(d * d, axis=0, keepdims=True)
    hn = d * lax.rsqrt(var + 1e-5)
    o_ref[...] = jnp.maximum(hn * g_ref[...] + be_ref[...], 0.0)


def kernel(X, edge_index, W_node, W_self, b_self, gamma, beta):
    x2 = X.reshape(V, F)
    xpad = jnp.concatenate([x2, jnp.zeros((8, F), x2.dtype)], axis=0)
    xsplit = jnp.stack([xpad[:, :FH], xpad[:, FH:]])
    pad_n = EPAD - E
    src = jnp.concatenate(
        [edge_index[:, 0], jnp.full((pad_n,), V, jnp.int32)])
    dst = jnp.concatenate(
        [edge_index[:, 1], jnp.zeros((pad_n,), jnp.int32)])
    src3 = src.reshape(NS, NCH, CH)
    dst3 = dst.reshape(NS, NCH, CH)
    zeros_vf = jnp.zeros((VP, FH), jnp.float32)

    scatter = pl.kernel(
        _sc_scatter_body,
        out_type=jax.ShapeDtypeStruct((NC, VP, FH), jnp.float32),
        mesh=plsc.VectorSubcoreMesh(core_axis_name="c", subcore_axis_name="s"),
        scratch_types=[
            pltpu.VMEM((NCH, CH), jnp.int32),
            pltpu.VMEM((NCH, CH), jnp.int32),
            pltpu.VMEM((NB, CH, FH), jnp.float32),
            pltpu.SemaphoreType.DMA,
            pltpu.SemaphoreType.DMA,
            pltpu.SemaphoreType.DMA,
            pltpu.SemaphoreType.DMA,
            pltpu.SemaphoreType.DMA,
            pltpu.SemaphoreType.DMA,
            pltpu.SemaphoreType.DMA,
            pltpu.SemaphoreType.DMA,
            pltpu.VMEM_SHARED((VP, FH), jnp.float32),
        ],
        compiler_params=pltpu.CompilerParams(use_tc_tiling_on_sc=False),
    )
    partials = scatter(xsplit, src3, dst3, zeros_vf)

    out2 = pl.pallas_call(
        _tc_combine_body,
        out_shape=jax.ShapeDtypeStruct((V, F), jnp.float32),
        grid=(1,),
        in_specs=[
            pl.BlockSpec((V, F), lambda i: (0, 0)),
            pl.BlockSpec((NC, V, FH), lambda i: (0, 0, 0)),
            pl.BlockSpec((F, F), lambda i: (0, 0)),
            pl.BlockSpec((F, F), lambda i: (0, 0)),
            pl.BlockSpec((1, F), lambda i: (0, 0)),
            pl.BlockSpec((1, F), lambda i: (0, 0)),
            pl.BlockSpec((1, F), lambda i: (0, 0)),
        ],
        out_specs=pl.BlockSpec((V, F), lambda i: (0, 0)),
    )(x2, partials, W_self.T, W_node.T, b_self.reshape(1, F),
      gamma.reshape(1, F), beta.reshape(1, F))
    return out2.reshape(1, V, F)


# speedup vs baseline: 1.0087x; 1.0087x over previous
import jax
import jax.numpy as jnp
import numpy as np
from jax import lax
from jax.experimental import pallas as pl
from jax.experimental.pallas import tpu as pltpu
from jax.experimental.pallas import tpu_sc as plsc

V = 10000
F = 128
FH = F // 2
E = 320000
NC = 2
NS = 16
CH = 128
NCH = 160
EPT = CH * NCH
EPAD = NS * EPT
VP = 10112
RPT = VP // NS


NB = 4
FW = FH // 2

_PERM = np.empty((FH,), np.int32)
for _h in (0, 1):
    for _k in range(16):
        _PERM[32 * _h + 2 * _k] = 32 * _h + _k
        _PERM[32 * _h + 2 * _k + 1] = 32 * _h + 16 + _k


def _sc_scatter_body(x_hbm, src_hbm, dst_hbm, zero_hbm, out_hbm,
                     src_v, dst_v, rows_bf, rows_f,
                     g0, g1, g2, g3, s0, s1, s2, s3, acc):
    c = lax.axis_index("c")
    s = lax.axis_index("s")
    gsems = (g0, g1, g2, g3)
    ssems = (s0, s1, s2, s3)
    xc = x_hbm.at[c]

    def slot(j, b, wait_s, issue_g, jm2=None, jp2=None):
        pltpu.make_async_copy(xc.at[src_v.at[j]], rows_bf.at[b],
                              gsems[b]).wait()

        @pl.loop(0, CH // 8)
        def _(ri):
            for u in range(8):
                r = ri * 8 + u
                for h in range(2):
                    w = rows_bf[b, r, pl.ds(h * 16, 16)]
                    lo, hi = plsc.unpack(
                        plsc.bitcast(w, jnp.bfloat16),
                        format=plsc.PackFormat.INTERLEAVED)
                    rows_f[b, r, pl.ds(32 * h, 16)] = lo
                    rows_f[b, r, pl.ds(32 * h + 16, 16)] = hi

        pltpu.async_copy(rows_f.at[b], acc.at[dst_v.at[j]], ssems[b],
                         add=True)
        b2 = (b + 2) % NB
        if wait_s:
            pltpu.make_async_copy(rows_f.at[b2], acc.at[dst_v.at[jm2]],
                                  ssems[b2]).wait()
        if issue_g:
            pltpu.async_copy(xc.at[src_v.at[jp2]], rows_bf.at[b2], gsems[b2])

    pltpu.sync_copy(zero_hbm.at[pl.ds(s * RPT, RPT)],
                    acc.at[pl.ds(s * RPT, RPT)])
    pltpu.sync_copy(src_hbm.at[s], src_v)
    pltpu.sync_copy(dst_hbm.at[s], dst_v)
    for b in range(2):
        pltpu.async_copy(xc.at[src_v.at[b]], rows_bf.at[b], gsems[b])
    plsc.subcore_barrier()

    for b in range(NB):
        slot(b, b, wait_s=(b >= 2), issue_g=True, jm2=b - 2, jp2=b + 2)

    @pl.loop(1, NCH // NB - 1)
    def _(i):
        for b in range(NB):
            j = i * NB + b
            slot(j, b, wait_s=True, issue_g=True, jm2=j - 2, jp2=j + 2)

    base = NCH - NB
    for b in range(NB):
        slot(base + b, b, wait_s=True, issue_g=(b < 2),
             jm2=base + b - 2, jp2=base + b + 2)
    for b in (2, 3):
        pltpu.make_async_copy(rows_f.at[b], acc.at[dst_v.at[base + b]],
                              ssems[b]).wait()
    plsc.subcore_barrier()
    pltpu.sync_copy(acc.at[pl.ds(s * RPT, RPT)],
                    out_hbm.at[c, pl.ds(s * RPT, RPT)])


def _tc_combine_body(x_ref, p_ref, wst_ref, wnt_ref, b_ref, g_ref, be_ref,
                     o_ref):
    x = x_ref[...]
    agg = jnp.concatenate([p_ref[0], p_ref[1]], axis=-1)
    h = (jnp.dot(x, wst_ref[...], preferred_element_type=jnp.float32)
         + jnp.dot(agg, wnt_ref[...], preferred_element_type=jnp.float32)
         + b_ref[...])
    mean = jnp.mean(h, axis=0, keepdims=True)
    d = h - mean
    var = jnp.mean(d * d, axis=0, keepdims=True)
    hn = d * lax.rsqrt(var + 1e-5)
    o_ref[...] = jnp.maximum(hn * g_ref[...] + be_ref[...], 0.0)


def kernel(X, edge_index, W_node, W_self, b_self, gamma, beta):
    x2 = X.reshape(V, F)
    xpad = jnp.concatenate([x2, jnp.zeros((8, F), x2.dtype)], axis=0)
    xb = xpad.astype(jnp.bfloat16)
    xp = jnp.stack([xb[:, :FH][:, _PERM], xb[:, FH:][:, _PERM]])
    xsplit = jax.lax.bitcast_convert_type(
        xp.reshape(NC, V + 8, FW, 2), jnp.int32)
    pad_n = EPAD - E
    src = jnp.concatenate(
        [edge_index[:, 0], jnp.full((pad_n,), V, jnp.int32)])
    dst = jnp.concatenate(
        [edge_index[:, 1], jnp.zeros((pad_n,), jnp.int32)])
    src3 = src.reshape(NS, NCH, CH)
    dst3 = dst.reshape(NS, NCH, CH)
    zeros_vf = jnp.zeros((VP, FH), jnp.float32)

    scatter = pl.kernel(
        _sc_scatter_body,
        out_type=jax.ShapeDtypeStruct((NC, VP, FH), jnp.float32),
        mesh=plsc.VectorSubcoreMesh(core_axis_name="c", subcore_axis_name="s"),
        scratch_types=[
            pltpu.VMEM((NCH, CH), jnp.int32),
            pltpu.VMEM((NCH, CH), jnp.int32),
            pltpu.VMEM((NB, CH, FW), jnp.int32),
            pltpu.VMEM((NB, CH, FH), jnp.float32),
            pltpu.SemaphoreType.DMA,
            pltpu.SemaphoreType.DMA,
            pltpu.SemaphoreType.DMA,
            pltpu.SemaphoreType.DMA,
            pltpu.SemaphoreType.DMA,
            pltpu.SemaphoreType.DMA,
            pltpu.SemaphoreType.DMA,
            pltpu.SemaphoreType.DMA,
            pltpu.VMEM_SHARED((VP, FH), jnp.float32),
        ],
        compiler_params=pltpu.CompilerParams(use_tc_tiling_on_sc=False,
                                             needs_layout_passes=False),
    )
    partials = scatter(xsplit, src3, dst3, zeros_vf)

    out2 = pl.pallas_call(
        _tc_combine_body,
        out_shape=jax.ShapeDtypeStruct((V, F), jnp.float32),
        grid=(1,),
        in_specs=[
            pl.BlockSpec((V, F), lambda i: (0, 0)),
            pl.BlockSpec((NC, V, FH), lambda i: (0, 0, 0)),
            pl.BlockSpec((F, F), lambda i: (0, 0)),
            pl.BlockSpec((F, F), lambda i: (0, 0)),
            pl.BlockSpec((1, F), lambda i: (0, 0)),
            pl.BlockSpec((1, F), lambda i: (0, 0)),
            pl.BlockSpec((1, F), lambda i: (0, 0)),
        ],
        out_specs=pl.BlockSpec((V, F), lambda i: (0, 0)),
    )(x2, partials, W_self.T, W_node.T, b_self.reshape(1, F),
      gamma.reshape(1, F), beta.reshape(1, F))
    return out2.reshape(1, V, F)


# speedup vs baseline: 1.2458x; 1.2351x over previous
import jax
import jax.numpy as jnp
import numpy as np
from jax import lax
from jax.experimental import pallas as pl
from jax.experimental.pallas import tpu as pltpu
from jax.experimental.pallas import tpu_sc as plsc

V = 10000
F = 128
FH = F // 2
E = 320000
NC = 2
NS = 16
CH = 128
NCH = 160
EPT = CH * NCH
EPAD = NS * EPT
VP = 10112
RPT = VP // NS


NB = 4
FW = FH // 2

_PERM = np.empty((FH,), np.int32)
for _h in (0, 1):
    for _k in range(16):
        _PERM[32 * _h + 2 * _k] = 32 * _h + _k
        _PERM[32 * _h + 2 * _k + 1] = 32 * _h + 16 + _k


def _sc_scatter_body(x_hbm, src_hbm, dst_hbm, zero_hbm, out_hbm,
                     src_v, dst_v, rows_bf, rows_f,
                     g0, g1, g2, g3, s0, s1, s2, s3, acc):
    c = lax.axis_index("c")
    s = lax.axis_index("s")
    gsems = (g0, g1, g2, g3)
    ssems = (s0, s1, s2, s3)
    xc = x_hbm.at[c]

    def slot(j, b, wait_s, issue_g, jm2=None, jp2=None):
        pltpu.make_async_copy(xc.at[src_v.at[j]], rows_bf.at[b],
                              gsems[b]).wait()

        @plsc.parallel_loop(0, CH, step=8, unroll=4)
        def _(r0):
            for u in range(8):
                r = r0 + u
                for h in range(2):
                    w = rows_bf[b, r, pl.ds(h * 16, 16)]
                    lo, hi = plsc.unpack(
                        plsc.bitcast(w, jnp.bfloat16),
                        format=plsc.PackFormat.INTERLEAVED)
                    rows_f[b, r, pl.ds(32 * h, 16)] = lo
                    rows_f[b, r, pl.ds(32 * h + 16, 16)] = hi

        pltpu.async_copy(rows_f.at[b], acc.at[dst_v.at[j]], ssems[b],
                         add=True)
        b2 = (b + 2) % NB
        if wait_s:
            pltpu.make_async_copy(rows_f.at[b2], acc.at[dst_v.at[jm2]],
                                  ssems[b2]).wait()
        if issue_g:
            pltpu.async_copy(xc.at[src_v.at[jp2]], rows_bf.at[b2], gsems[b2])

    pltpu.sync_copy(zero_hbm.at[pl.ds(s * RPT, RPT)],
                    acc.at[pl.ds(s * RPT, RPT)])
    pltpu.sync_copy(src_hbm.at[s], src_v)
    pltpu.sync_copy(dst_hbm.at[s], dst_v)
    for b in range(2):
        pltpu.async_copy(xc.at[src_v.at[b]], rows_bf.at[b], gsems[b])
    plsc.subcore_barrier()

    for b in range(NB):
        slot(b, b, wait_s=(b >= 2), issue_g=True, jm2=b - 2, jp2=b + 2)

    @pl.loop(1, NCH // NB - 1)
    def _(i):
        for b in range(NB):
            j = i * NB + b
            slot(j, b, wait_s=True, issue_g=True, jm2=j - 2, jp2=j + 2)

    base = NCH - NB
    for b in range(NB):
        slot(base + b, b, wait_s=True, issue_g=(b < 2),
             jm2=base + b - 2, jp2=base + b + 2)
    for b in (2, 3):
        pltpu.make_async_copy(rows_f.at[b], acc.at[dst_v.at[base + b]],
                              ssems[b]).wait()
    plsc.subcore_barrier()
    pltpu.sync_copy(acc.at[pl.ds(s * RPT, RPT)],
                    out_hbm.at[c, pl.ds(s * RPT, RPT)])


def _tc_combine_body(x_ref, p_ref, wst_ref, wnt_ref, b_ref, g_ref, be_ref,
                     o_ref):
    x = x_ref[...]
    agg = jnp.concatenate([p_ref[0], p_ref[1]], axis=-1)
    h = (jnp.dot(x, wst_ref[...], preferred_element_type=jnp.float32)
         + jnp.dot(agg, wnt_ref[...], preferred_element_type=jnp.float32)
         + b_ref[...])
    mean = jnp.mean(h, axis=0, keepdims=True)
    d = h - mean
    var = jnp.mean(d * d, axis=0, keepdims=True)
    hn = d * lax.rsqrt(var + 1e-5)
    o_ref[...] = jnp.maximum(hn * g_ref[...] + be_ref[...], 0.0)


def kernel(X, edge_index, W_node, W_self, b_self, gamma, beta):
    x2 = X.reshape(V, F)
    xpad = jnp.concatenate([x2, jnp.zeros((8, F), x2.dtype)], axis=0)
    xb = xpad.astype(jnp.bfloat16)
    xp = jnp.stack([xb[:, :FH][:, _PERM], xb[:, FH:][:, _PERM]])
    xsplit = jax.lax.bitcast_convert_type(
        xp.reshape(NC, V + 8, FW, 2), jnp.int32)
    pad_n = EPAD - E
    src = jnp.concatenate(
        [edge_index[:, 0], jnp.full((pad_n,), V, jnp.int32)])
    dst = jnp.concatenate(
        [edge_index[:, 1], jnp.zeros((pad_n,), jnp.int32)])
    src3 = src.reshape(NS, NCH, CH)
    dst3 = dst.reshape(NS, NCH, CH)
    zeros_vf = jnp.zeros((VP, FH), jnp.float32)

    scatter = pl.kernel(
        _sc_scatter_body,
        out_type=jax.ShapeDtypeStruct((NC, VP, FH), jnp.float32),
        mesh=plsc.VectorSubcoreMesh(core_axis_name="c", subcore_axis_name="s"),
        scratch_types=[
            pltpu.VMEM((NCH, CH), jnp.int32),
            pltpu.VMEM((NCH, CH), jnp.int32),
            pltpu.VMEM((NB, CH, FW), jnp.int32),
            pltpu.VMEM((NB, CH, FH), jnp.float32),
            pltpu.SemaphoreType.DMA,
            pltpu.SemaphoreType.DMA,
            pltpu.SemaphoreType.DMA,
            pltpu.SemaphoreType.DMA,
            pltpu.SemaphoreType.DMA,
            pltpu.SemaphoreType.DMA,
            pltpu.SemaphoreType.DMA,
            pltpu.SemaphoreType.DMA,
            pltpu.VMEM_SHARED((VP, FH), jnp.float32),
        ],
        compiler_params=pltpu.CompilerParams(use_tc_tiling_on_sc=False,
                                             needs_layout_passes=False),
    )
    partials = scatter(xsplit, src3, dst3, zeros_vf)

    out2 = pl.pallas_call(
        _tc_combine_body,
        out_shape=jax.ShapeDtypeStruct((V, F), jnp.float32),
        grid=(1,),
        in_specs=[
            pl.BlockSpec((V, F), lambda i: (0, 0)),
            pl.BlockSpec((NC, V, FH), lambda i: (0, 0, 0)),
            pl.BlockSpec((F, F), lambda i: (0, 0)),
            pl.BlockSpec((F, F), lambda i: (0, 0)),
            pl.BlockSpec((1, F), lambda i: (0, 0)),
            pl.BlockSpec((1, F), lambda i: (0, 0)),
            pl.BlockSpec((1, F), lambda i: (0, 0)),
        ],
        out_specs=pl.BlockSpec((V, F), lambda i: (0, 0)),
    )(x2, partials, W_self.T, W_node.T, b_self.reshape(1, F),
      gamma.reshape(1, F), beta.reshape(1, F))
    return out2.reshape(1, V, F)


# speedup vs baseline: 1.3171x; 1.0572x over previous
import jax
import jax.numpy as jnp
from jax import lax
from jax.experimental import pallas as pl
from jax.experimental.pallas import tpu as pltpu
from jax.experimental.pallas import tpu_sc as plsc

V = 10000
F = 128
FH = F // 2
E = 320000
NC = 2
NS = 16
CH = 128
NCH = 160
EPT = CH * NCH
EPAD = NS * EPT
VP = 10112
RPT = VP // NS


NB = 4
FW = FH // 2

def _tc_pack_body(x_ref, o_ref):
    x = x_ref[...]

    def pack_half(xh):
        lo = jnp.concatenate([xh[:, 0:16], xh[:, 32:48]], axis=1)
        hi = jnp.concatenate([xh[:, 16:32], xh[:, 48:64]], axis=1)
        lo_u = jax.lax.bitcast_convert_type(lo, jnp.uint32)
        hi_u = jax.lax.bitcast_convert_type(hi, jnp.uint32)
        lo_b = (lo_u + jnp.uint32(0x8000)) >> jnp.uint32(16)
        hi_b = (hi_u + jnp.uint32(0x8000)) & jnp.uint32(0xFFFF0000)
        return jax.lax.bitcast_convert_type(lo_b | hi_b, jnp.int32)

    o_ref[0] = pack_half(x[:, :FH])
    o_ref[1] = pack_half(x[:, FH:])


def _sc_scatter_body(x_hbm, src_hbm, dst_hbm, out_hbm,
                     src_v, dst_v, rows_bf, rows_f,
                     g0, g1, g2, g3, s0, s1, s2, s3, acc):
    c = lax.axis_index("c")
    s = lax.axis_index("s")
    gsems = (g0, g1, g2, g3)
    ssems = (s0, s1, s2, s3)
    xc = x_hbm.at[c]

    def slot(j, b, wait_s, issue_g, jm2=None, jp2=None):
        pltpu.make_async_copy(xc.at[src_v.at[j]], rows_bf.at[b],
                              gsems[b]).wait()

        @plsc.parallel_loop(0, CH, step=8, unroll=4)
        def _(r0):
            for u in range(8):
                r = r0 + u
                for h in range(2):
                    w = rows_bf[b, r, pl.ds(h * 16, 16)]
                    lo, hi = plsc.unpack(
                        plsc.bitcast(w, jnp.bfloat16),
                        format=plsc.PackFormat.INTERLEAVED)
                    rows_f[b, r, pl.ds(32 * h, 16)] = lo
                    rows_f[b, r, pl.ds(32 * h + 16, 16)] = hi

        pltpu.async_copy(rows_f.at[b], acc.at[dst_v.at[j]], ssems[b],
                         add=True)
        b2 = (b + 2) % NB
        if wait_s:
            pltpu.make_async_copy(rows_f.at[b2], acc.at[dst_v.at[jm2]],
                                  ssems[b2]).wait()
        if issue_g:
            pltpu.async_copy(xc.at[src_v.at[jp2]], rows_bf.at[b2], gsems[b2])

    @pl.loop(0, CH)
    def _(r):
        for q in range(FH // 16):
            rows_f[0, r, pl.ds(16 * q, 16)] = jnp.zeros((16,), jnp.float32)
    for q in range(4):
        pltpu.sync_copy(rows_f.at[0],
                        acc.at[pl.ds(s * RPT + q * CH, CH)])
    pltpu.sync_copy(rows_f.at[0, pl.ds(0, RPT - 4 * CH)],
                    acc.at[pl.ds(s * RPT + 4 * CH, RPT - 4 * CH)])
    pltpu.sync_copy(src_hbm.at[s], src_v)
    pltpu.sync_copy(dst_hbm.at[s], dst_v)
    for b in range(2):
        pltpu.async_copy(xc.at[src_v.at[b]], rows_bf.at[b], gsems[b])
    plsc.subcore_barrier()

    for b in range(NB):
        slot(b, b, wait_s=(b >= 2), issue_g=True, jm2=b - 2, jp2=b + 2)

    @pl.loop(1, NCH // NB - 1)
    def _(i):
        for b in range(NB):
            j = i * NB + b
            slot(j, b, wait_s=True, issue_g=True, jm2=j - 2, jp2=j + 2)

    base = NCH - NB
    for b in range(NB):
        slot(base + b, b, wait_s=True, issue_g=(b < 2),
             jm2=base + b - 2, jp2=base + b + 2)
    for b in (2, 3):
        pltpu.make_async_copy(rows_f.at[b], acc.at[dst_v.at[base + b]],
                              ssems[b]).wait()
    plsc.subcore_barrier()
    pltpu.sync_copy(acc.at[pl.ds(s * RPT, RPT)],
                    out_hbm.at[c, pl.ds(s * RPT, RPT)])


def _tc_combine_body(x_ref, p_ref, ws_ref, wn_ref, b_ref, g_ref, be_ref,
                     o_ref):
    x = x_ref[...]
    agg = jnp.concatenate([p_ref[0], p_ref[1]], axis=-1)
    dn = (((1,), (1,)), ((), ()))
    h = (lax.dot_general(x, ws_ref[...], dn,
                         preferred_element_type=jnp.float32)
         + lax.dot_general(agg, wn_ref[...], dn,
                           preferred_element_type=jnp.float32)
         + b_ref[...])
    mean = jnp.mean(h, axis=0, keepdims=True)
    d = h - mean
    var = jnp.mean(d * d, axis=0, keepdims=True)
    hn = d * lax.rsqrt(var + 1e-5)
    o_ref[...] = jnp.maximum(hn * g_ref[...] + be_ref[...], 0.0)


def kernel(X, edge_index, W_node, W_self, b_self, gamma, beta):
    x2 = X.reshape(V, F)
    xpad = jnp.concatenate([x2, jnp.zeros((8, F), x2.dtype)], axis=0)
    xsplit = pl.pallas_call(
        _tc_pack_body,
        out_shape=jax.ShapeDtypeStruct((NC, V + 8, FW), jnp.int32),
    )(xpad)
    pad_n = EPAD - E
    src = jnp.concatenate(
        [edge_index[:, 0], jnp.full((pad_n,), V, jnp.int32)])
    dst = jnp.concatenate(
        [edge_index[:, 1], jnp.zeros((pad_n,), jnp.int32)])
    src3 = src.reshape(NS, NCH, CH)
    dst3 = dst.reshape(NS, NCH, CH)

    scatter = pl.kernel(
        _sc_scatter_body,
        out_type=jax.ShapeDtypeStruct((NC, VP, FH), jnp.float32),
        mesh=plsc.VectorSubcoreMesh(core_axis_name="c", subcore_axis_name="s"),
        scratch_types=[
            pltpu.VMEM((NCH, CH), jnp.int32),
            pltpu.VMEM((NCH, CH), jnp.int32),
            pltpu.VMEM((NB, CH, FW), jnp.int32),
            pltpu.VMEM((NB, CH, FH), jnp.float32),
            pltpu.SemaphoreType.DMA,
            pltpu.SemaphoreType.DMA,
            pltpu.SemaphoreType.DMA,
            pltpu.SemaphoreType.DMA,
            pltpu.SemaphoreType.DMA,
            pltpu.SemaphoreType.DMA,
            pltpu.SemaphoreType.DMA,
            pltpu.SemaphoreType.DMA,
            pltpu.VMEM_SHARED((VP, FH), jnp.float32),
        ],
        compiler_params=pltpu.CompilerParams(use_tc_tiling_on_sc=False,
                                             needs_layout_passes=False),
    )
    partials = scatter(xsplit, src3, dst3)

    out2 = pl.pallas_call(
        _tc_combine_body,
        out_shape=jax.ShapeDtypeStruct((V, F), jnp.float32),
        grid=(1,),
        in_specs=[
            pl.BlockSpec((V, F), lambda i: (0, 0)),
            pl.BlockSpec((NC, V, FH), lambda i: (0, 0, 0)),
            pl.BlockSpec((F, F), lambda i: (0, 0)),
            pl.BlockSpec((F, F), lambda i: (0, 0)),
            pl.BlockSpec((1, F), lambda i: (0, 0)),
            pl.BlockSpec((1, F), lambda i: (0, 0)),
            pl.BlockSpec((1, F), lambda i: (0, 0)),
        ],
        out_specs=pl.BlockSpec((V, F), lambda i: (0, 0)),
    )(x2, partials, W_self, W_node, b_self.reshape(1, F),
      gamma.reshape(1, F), beta.reshape(1, F))
    return out2.reshape(1, V, F)


# speedup vs baseline: 1.3946x; 1.0588x over previous
import jax
import jax.numpy as jnp
from jax import lax
from jax.experimental import pallas as pl
from jax.experimental.pallas import tpu as pltpu
from jax.experimental.pallas import tpu_sc as plsc

V = 10000
F = 128
FH = F // 2
E = 320000
NC = 2
NS = 16
CH = 128
NCH = 160
EPT = CH * NCH
EPAD = NS * EPT
VP = 10112
RPT = VP // NS


NB = 4
FW = FH // 2

def _tc_pack_body(x_ref, o_ref):
    x = x_ref[...]

    def pack_half(xh):
        lo = jnp.concatenate([xh[:, 0:16], xh[:, 32:48]], axis=1)
        hi = jnp.concatenate([xh[:, 16:32], xh[:, 48:64]], axis=1)
        lo_u = jax.lax.bitcast_convert_type(lo, jnp.uint32)
        hi_u = jax.lax.bitcast_convert_type(hi, jnp.uint32)
        lo_b = (lo_u + jnp.uint32(0x8000)) >> jnp.uint32(16)
        hi_b = (hi_u + jnp.uint32(0x8000)) & jnp.uint32(0xFFFF0000)
        return jax.lax.bitcast_convert_type(lo_b | hi_b, jnp.int32)

    o_ref[0] = pack_half(x[:, :FH])
    o_ref[1] = pack_half(x[:, FH:])


def _sc_scatter_body(x_hbm, src_hbm, dst_hbm, out_hbm,
                     src_v, dst_v, rows_bf, rows_f,
                     g0, g1, g2, g3, s0, s1, s2, s3, acc):
    c = lax.axis_index("c")
    s = lax.axis_index("s")
    gsems = (g0, g1, g2, g3)
    ssems = (s0, s1, s2, s3)
    xc = x_hbm.at[c]

    def slot(j, b, wait_s, issue_g, jm2=None, jp2=None):
        pltpu.make_async_copy(xc.at[src_v.at[j]], rows_bf.at[b],
                              gsems[b]).wait()

        @plsc.parallel_loop(0, CH, step=8, unroll=4)
        def _(r0):
            for u in range(8):
                r = r0 + u
                for h in range(2):
                    w = rows_bf[b, r, pl.ds(h * 16, 16)]
                    lo, hi = plsc.unpack(
                        plsc.bitcast(w, jnp.bfloat16),
                        format=plsc.PackFormat.INTERLEAVED)
                    rows_f[b, r, pl.ds(32 * h, 16)] = lo
                    rows_f[b, r, pl.ds(32 * h + 16, 16)] = hi

        pltpu.async_copy(rows_f.at[b], acc.at[dst_v.at[j]], ssems[b],
                         add=True)
        b2 = (b + 2) % NB
        if wait_s:
            pltpu.make_async_copy(rows_f.at[b2], acc.at[dst_v.at[jm2]],
                                  ssems[b2]).wait()
        if issue_g:
            pltpu.async_copy(xc.at[src_v.at[jp2]], rows_bf.at[b2], gsems[b2])

    @pl.loop(0, CH)
    def _(r):
        for q in range(FH // 16):
            rows_f[0, r, pl.ds(16 * q, 16)] = jnp.zeros((16,), jnp.float32)
    for q in range(4):
        pltpu.sync_copy(rows_f.at[0],
                        acc.at[pl.ds(s * RPT + q * CH, CH)])
    pltpu.sync_copy(rows_f.at[0, pl.ds(0, RPT - 4 * CH)],
                    acc.at[pl.ds(s * RPT + 4 * CH, RPT - 4 * CH)])
    pltpu.sync_copy(src_hbm.at[s], src_v)
    pltpu.sync_copy(dst_hbm.at[s], dst_v)
    for b in range(2):
        pltpu.async_copy(xc.at[src_v.at[b]], rows_bf.at[b], gsems[b])
    plsc.subcore_barrier()

    for b in range(NB):
        slot(b, b, wait_s=(b >= 2), issue_g=True, jm2=b - 2, jp2=b + 2)

    @pl.loop(1, NCH // NB - 1)
    def _(i):
        for b in range(NB):
            j = i * NB + b
            slot(j, b, wait_s=True, issue_g=True, jm2=j - 2, jp2=j + 2)

    base = NCH - NB
    for b in range(NB):
        slot(base + b, b, wait_s=True, issue_g=(b < 2),
             jm2=base + b - 2, jp2=base + b + 2)
    for b in (2, 3):
        pltpu.make_async_copy(rows_f.at[b], acc.at[dst_v.at[base + b]],
                              ssems[b]).wait()
    plsc.subcore_barrier()
    pltpu.sync_copy(acc.at[pl.ds(s * RPT, RPT)],
                    out_hbm.at[c, pl.ds(s * RPT, RPT)])


def _tc_combine_body(x_ref, p_ref, ws_ref, wn_ref, b_ref, g_ref, be_ref,
                     o_ref):
    x = x_ref[...]
    agg = jnp.concatenate([p_ref[0], p_ref[1]], axis=-1)
    dn = (((1,), (1,)), ((), ()))
    h = (lax.dot_general(x, ws_ref[...], dn,
                         preferred_element_type=jnp.float32)
         + lax.dot_general(agg, wn_ref[...], dn,
                           preferred_element_type=jnp.float32)
         + b_ref[...])
    mean = jnp.mean(h, axis=0, keepdims=True)
    d = h - mean
    var = jnp.mean(d * d, axis=0, keepdims=True)
    hn = d * lax.rsqrt(var + 1e-5)
    o_ref[...] = jnp.maximum(hn * g_ref[...] + be_ref[...], 0.0)


def kernel(X, edge_index, W_node, W_self, b_self, gamma, beta):
    x2 = X.reshape(V, F)
    xpad = jnp.concatenate([x2, jnp.zeros((8, F), x2.dtype)], axis=0)
    xsplit = pl.pallas_call(
        _tc_pack_body,
        out_shape=jax.ShapeDtypeStruct((NC, V + 8, FW), jnp.int32),
        grid=(9,),
        in_specs=[pl.BlockSpec(((V + 8) // 9, F), lambda i: (i, 0))],
        out_specs=pl.BlockSpec((NC, (V + 8) // 9, FW), lambda i: (0, i, 0)),
    )(xpad)
    pad_n = EPAD - E
    src = jnp.concatenate(
        [edge_index[:, 0], jnp.full((pad_n,), V, jnp.int32)])
    dst = jnp.concatenate(
        [edge_index[:, 1], jnp.zeros((pad_n,), jnp.int32)])
    src3 = src.reshape(NS, NCH, CH)
    dst3 = dst.reshape(NS, NCH, CH)

    scatter = pl.kernel(
        _sc_scatter_body,
        out_type=jax.ShapeDtypeStruct((NC, VP, FH), jnp.float32),
        mesh=plsc.VectorSubcoreMesh(core_axis_name="c", subcore_axis_name="s"),
        scratch_types=[
            pltpu.VMEM((NCH, CH), jnp.int32),
            pltpu.VMEM((NCH, CH), jnp.int32),
            pltpu.VMEM((NB, CH, FW), jnp.int32),
            pltpu.VMEM((NB, CH, FH), jnp.float32),
            pltpu.SemaphoreType.DMA,
            pltpu.SemaphoreType.DMA,
            pltpu.SemaphoreType.DMA,
            pltpu.SemaphoreType.DMA,
            pltpu.SemaphoreType.DMA,
            pltpu.SemaphoreType.DMA,
            pltpu.SemaphoreType.DMA,
            pltpu.SemaphoreType.DMA,
            pltpu.VMEM_SHARED((VP, FH), jnp.float32),
        ],
        compiler_params=pltpu.CompilerParams(use_tc_tiling_on_sc=False,
                                             needs_layout_passes=False),
    )
    partials = scatter(xsplit, src3, dst3)

    out2 = pl.pallas_call(
        _tc_combine_body,
        out_shape=jax.ShapeDtypeStruct((V, F), jnp.float32),
        grid=(1,),
        in_specs=[
            pl.BlockSpec((V, F), lambda i: (0, 0)),
            pl.BlockSpec((NC, V, FH), lambda i: (0, 0, 0)),
            pl.BlockSpec((F, F), lambda i: (0, 0)),
            pl.BlockSpec((F, F), lambda i: (0, 0)),
            pl.BlockSpec((1, F), lambda i: (0, 0)),
            pl.BlockSpec((1, F), lambda i: (0, 0)),
            pl.BlockSpec((1, F), lambda i: (0, 0)),
        ],
        out_specs=pl.BlockSpec((V, F), lambda i: (0, 0)),
    )(x2, partials, W_self, W_node, b_self.reshape(1, F),
      gamma.reshape(1, F), beta.reshape(1, F))
    return out2.reshape(1, V, F)
